# baseline (speedup 1.0000x reference)
import functools

import jax
import jax.numpy as jnp
from jax import lax
from jax.experimental import pallas as pl
from jax.experimental.pallas import tpu as pltpu
from jax.experimental.pallas import tpu_sc as plsc

B = 16384
L = 200
VOCAB = 769
NC = 2
NS = 16
NW = NC * NS
BPW = B // NW
CR = 128
NCH = BPW // CR
ROWS_PER_GROUP = 16
NGROUPS = CR // ROWS_PER_GROUP


def _row_sum(xb_v, b, r, tab_v, tail_mask):
    idx = xb_v[b, r, pl.ds(0, 16)]
    acc0 = plsc.load_gather(tab_v, [idx])
    idx = xb_v[b, r, pl.ds(16, 16)]
    acc1 = plsc.load_gather(tab_v, [idx])
    for j in range(2, 12):
        idx = xb_v[b, r, pl.ds(16 * j, 16)]
        g = plsc.load_gather(tab_v, [idx])
        if j % 2 == 0:
            acc0 = acc0 + g
        else:
            acc1 = acc1 + g
    idx = xb_v[b, r, pl.ds(L - 16, 16)]
    v = plsc.load_gather(tab_v, [idx])
    acc0 = acc0 + jnp.where(tail_mask, v, 0.0)
    return jnp.sum(acc0 + acc1)


def _pst_kernel(x_hbm, tab_hbm, out_hbm, tab_v, xb_v, out_v, sem0, sem1):
    wid = lax.axis_index("s") * NC + lax.axis_index("c")
    row0 = wid * BPW
    pltpu.sync_copy(tab_hbm, tab_v)

    sems = (sem0, sem1)
    copies = [None, None]
    for c in range(min(2, NCH)):
        copies[c] = pltpu.async_copy(
            x_hbm.at[pl.ds(row0 + c * CR, CR)], xb_v.at[c], sems[c]
        )

    lane = lax.iota(jnp.int32, 16)
    tail_mask = lane >= 8

    for c in range(NCH):
        b = c % 2
        copies[b].wait()

        @plsc.parallel_loop(0, NGROUPS, 1)
        def group_body(g16, b=b, c=c):
            outv = jnp.zeros((16,), jnp.float32)
            for g in range(ROWS_PER_GROUP):
                r = g16 * ROWS_PER_GROUP + g
                rs = _row_sum(xb_v, b, r, tab_v, tail_mask)
                outv = jnp.where(lane == g, rs, outv)
            out_v[pl.ds(c * CR + g16 * ROWS_PER_GROUP, 16)] = outv

        if c + 2 < NCH:
            copies[b] = pltpu.async_copy(
                x_hbm.at[pl.ds(row0 + (c + 2) * CR, CR)], xb_v.at[b], sems[b]
            )

    pltpu.sync_copy(out_v, out_hbm.at[pl.ds(row0, BPW)])


@jax.jit
def _pst_sum(x, tab_flat):
    mesh = plsc.VectorSubcoreMesh(core_axis_name="c", subcore_axis_name="s")
    f = pl.kernel(
        _pst_kernel,
        out_type=jax.ShapeDtypeStruct((B,), jnp.float32),
        mesh=mesh,
        scratch_types=[
            pltpu.VMEM((VOCAB,), jnp.float32),
            pltpu.VMEM((2, CR, L), jnp.int32),
            pltpu.VMEM((BPW,), jnp.float32),
            pltpu.SemaphoreType.DMA,
            pltpu.SemaphoreType.DMA,
        ],
        compiler_params=pltpu.CompilerParams(needs_layout_passes=False),
    )
    return f(x, tab_flat)


def kernel(x, pst_weight, emb_weight):
    return _pst_sum(x.astype(jnp.int32), pst_weight.reshape(-1))


# speedup vs baseline: 1.2165x; 1.2165x over previous
import functools

import jax
import jax.numpy as jnp
from jax import lax
from jax.experimental import pallas as pl
from jax.experimental.pallas import tpu as pltpu
from jax.experimental.pallas import tpu_sc as plsc

B = 16384
L = 200
VOCAB = 769
NC = 2
NS = 16
NW = NC * NS
BPW = B // NW
CC = 128
NCH = BPW // CC
GRP = 16
NGROUPS = CC // GRP
NACC = 4


def _pst_kernel(xt_hbm, tab_hbm, out_hbm, tab_v, xb_v, out_v, sem0, sem1):
    wid = lax.axis_index("s") * NC + lax.axis_index("c")
    col0 = wid * BPW
    pltpu.sync_copy(tab_hbm, tab_v)

    sems = (sem0, sem1)
    copies = [None, None]
    for c in range(min(2, NCH)):
        copies[c] = pltpu.async_copy(
            xt_hbm.at[:, pl.ds(col0 + c * CC, CC)], xb_v.at[c], sems[c]
        )

    for c in range(NCH):
        b = c % 2
        copies[b].wait()

        @plsc.parallel_loop(0, NGROUPS, 1)
        def group_body(g16, b=b, c=c):
            cbase = g16 * GRP
            accs = []
            for a in range(NACC):
                idx = xb_v[b, a, pl.ds(cbase, GRP)]
                accs.append(plsc.load_gather(tab_v, [idx]))
            for l in range(NACC, L):
                idx = xb_v[b, l, pl.ds(cbase, GRP)]
                g = plsc.load_gather(tab_v, [idx])
                accs[l % NACC] = accs[l % NACC] + g
            out_v[pl.ds(c * CC + cbase, GRP)] = (
                (accs[0] + accs[1]) + (accs[2] + accs[3])
            )

        if c + 2 < NCH:
            copies[b] = pltpu.async_copy(
                xt_hbm.at[:, pl.ds(col0 + (c + 2) * CC, CC)], xb_v.at[b], sems[b]
            )

    pltpu.sync_copy(out_v, out_hbm.at[pl.ds(col0, BPW)])


@jax.jit
def _pst_sum(xt, tab_flat):
    mesh = plsc.VectorSubcoreMesh(core_axis_name="c", subcore_axis_name="s")
    f = pl.kernel(
        _pst_kernel,
        out_type=jax.ShapeDtypeStruct((B,), jnp.float32),
        mesh=mesh,
        scratch_types=[
            pltpu.VMEM((VOCAB,), jnp.float32),
            pltpu.VMEM((2, L, CC), jnp.int32),
            pltpu.VMEM((BPW,), jnp.float32),
            pltpu.SemaphoreType.DMA,
            pltpu.SemaphoreType.DMA,
        ],
        compiler_params=pltpu.CompilerParams(needs_layout_passes=False),
    )
    return f(xt, tab_flat)


def kernel(x, pst_weight, emb_weight):
    xt = x.astype(jnp.int32).T
    return _pst_sum(xt, pst_weight.reshape(-1))


# speedup vs baseline: 2.1885x; 1.7990x over previous
import functools

import jax
import jax.numpy as jnp
from jax import lax
from jax.experimental import pallas as pl
from jax.experimental.pallas import tpu as pltpu
from jax.experimental.pallas import tpu_sc as plsc

B = 16384
L = 200
VOCAB = 769
NC = 2
NS = 16
NW = NC * NS
BPW = B // NW
CC = 128
NCH = BPW // CC
GRP = 16
NGROUPS = CC // GRP
NACC = 4


def _pst_kernel(xt_hbm, tab_hbm, out_hbm, tab_v, xb_v, out_v, sem0, sem1):
    wid = lax.axis_index("s") * NC + lax.axis_index("c")
    col0 = wid * BPW
    pltpu.sync_copy(tab_hbm, tab_v)

    sems = (sem0, sem1)
    copies = [None, None]
    for c in range(min(2, NCH)):
        copies[c] = pltpu.async_copy(
            xt_hbm.at[:, pl.ds(col0 + c * CC, CC)], xb_v.at[c], sems[c]
        )

    zero = jnp.zeros((GRP,), jnp.float32)

    for c in range(NCH):
        b = c % 2
        copies[b].wait()

        def group_body(g16, _, b=b, c=c):
            cbase = g16 * GRP

            @plsc.parallel_loop(0, L, NACC, unroll=2, carry=(zero,) * NACC)
            def l_loop(l, accs, b=b, cbase=cbase):
                new = []
                for k in range(NACC):
                    idx = xb_v[b, l + k, pl.ds(cbase, GRP)]
                    new.append(accs[k] + plsc.load_gather(tab_v, [idx]))
                return tuple(new)

            a0, a1, a2, a3 = l_loop
            out_v[pl.ds(c * CC + cbase, GRP)] = (a0 + a1) + (a2 + a3)
            return _

        lax.fori_loop(0, NGROUPS, group_body, None)

        if c + 2 < NCH:
            copies[b] = pltpu.async_copy(
                xt_hbm.at[:, pl.ds(col0 + (c + 2) * CC, CC)], xb_v.at[b], sems[b]
            )

    pltpu.sync_copy(out_v, out_hbm.at[pl.ds(col0, BPW)])


@jax.jit
def _pst_sum(xt, tab_flat):
    mesh = plsc.VectorSubcoreMesh(core_axis_name="c", subcore_axis_name="s")
    f = pl.kernel(
        _pst_kernel,
        out_type=jax.ShapeDtypeStruct((B,), jnp.float32),
        mesh=mesh,
        scratch_types=[
            pltpu.VMEM((VOCAB,), jnp.float32),
            pltpu.VMEM((2, L, CC), jnp.int32),
            pltpu.VMEM((BPW,), jnp.float32),
            pltpu.SemaphoreType.DMA,
            pltpu.SemaphoreType.DMA,
        ],
        compiler_params=pltpu.CompilerParams(needs_layout_passes=False),
    )
    return f(xt, tab_flat)


def kernel(x, pst_weight, emb_weight):
    xt = x.astype(jnp.int32).T
    return _pst_sum(xt, pst_weight.reshape(-1))
